# Initial kernel scaffold; baseline (speedup 1.0000x reference)
import jax
import jax.numpy as jnp
from jax.experimental import pallas as pl


def kernel(x, edge_index, edge_attr, batch, W1, b1, W2, b2, W3, b3, Wfc, bfc):
    raise NotImplementedError("write your pallas kernel here")


# speedup vs baseline: 1.0109x; 1.0109x over previous
import functools

import jax
import jax.numpy as jnp
from jax.experimental import pallas as pl
from jax.experimental.pallas import tpu as pltpu

_N = 50000
_E = 800000
_G = 64
_BR = 512
_NPAD = 50176


def _mm_kernel(x_ref, w_ref, o_ref):
    o_ref[...] = jnp.dot(x_ref[...], w_ref[...],
                         preferred_element_type=jnp.float32)


def _mm(x, w):
    m, k = x.shape
    n = w.shape[1]
    grid = (m // _BR,)
    return pl.pallas_call(
        _mm_kernel,
        grid=grid,
        in_specs=[
            pl.BlockSpec((_BR, k), lambda i: (i, 0)),
            pl.BlockSpec((k, n), lambda i: (0, 0)),
        ],
        out_specs=pl.BlockSpec((_BR, n), lambda i: (i, 0)),
        out_shape=jax.ShapeDtypeStruct((m, n), jnp.float32),
    )(x, w)


def _fused_kernel(a_ref, b_ref, w_ref, o_ref):
    h = jnp.maximum(a_ref[...] + b_ref[0:1, :], 0.0)
    o_ref[...] = jnp.dot(h, w_ref[...], preferred_element_type=jnp.float32)


def _fused_mm(agg, b, w):
    m, k = agg.shape
    n = w.shape[1]
    b2 = jnp.broadcast_to(b[None, :], (8, k))
    return pl.pallas_call(
        _fused_kernel,
        grid=(m // _BR,),
        in_specs=[
            pl.BlockSpec((_BR, k), lambda i: (i, 0)),
            pl.BlockSpec((8, k), lambda i: (0, 0)),
            pl.BlockSpec((k, n), lambda i: (0, 0)),
        ],
        out_specs=pl.BlockSpec((_BR, n), lambda i: (i, 0)),
        out_shape=jax.ShapeDtypeStruct((m, n), jnp.float32),
    )(agg, b2, w)


def _pool_kernel(a_ref, b_ref, ids_ref, wfc_ref, bfc_ref, o_ref,
                 sums_ref, cnt_ref):
    step = pl.program_id(0)
    nsteps = pl.num_programs(0)

    @pl.when(step == 0)
    def _init():
        sums_ref[...] = jnp.zeros_like(sums_ref)
        cnt_ref[...] = jnp.zeros_like(cnt_ref)

    h3 = jnp.maximum(a_ref[...] + b_ref[0:1, :], 0.0)
    ids = ids_ref[0]
    rows = jax.lax.broadcasted_iota(jnp.int32, (_G, _BR), 0)
    onehot = (ids == rows).astype(jnp.float32)
    sums_ref[...] += jnp.dot(onehot, h3, preferred_element_type=jnp.float32)
    cnt_ref[...] += jnp.broadcast_to(
        jnp.sum(onehot, axis=1, keepdims=True), (_G, 128))

    @pl.when(step == nsteps - 1)
    def _fin():
        h4 = sums_ref[...] / jnp.maximum(cnt_ref[:, 0:1], 1.0)
        logits = jnp.dot(h4, wfc_ref[...],
                         preferred_element_type=jnp.float32) + bfc_ref[0:1, :]
        o_ref[...] = jax.nn.sigmoid(logits)


def _pool_fc(agg3, b3, batch_p, wfc, bfc):
    m, h3 = agg3.shape
    ids3d = batch_p.reshape(m // _BR, 1, _BR)
    b2 = jnp.broadcast_to(b3[None, :], (8, h3))
    wfc_p = jnp.pad(wfc, ((0, 0), (0, 127)))
    bfc_p = jnp.broadcast_to(bfc[0], (8, 128))
    out = pl.pallas_call(
        _pool_kernel,
        grid=(m // _BR,),
        in_specs=[
            pl.BlockSpec((_BR, h3), lambda i: (i, 0)),
            pl.BlockSpec((8, h3), lambda i: (0, 0)),
            pl.BlockSpec((1, 1, _BR), lambda i: (i, 0, 0)),
            pl.BlockSpec((h3, 128), lambda i: (0, 0)),
            pl.BlockSpec((8, 128), lambda i: (0, 0)),
        ],
        out_specs=pl.BlockSpec((_G, 128), lambda i: (0, 0)),
        out_shape=jax.ShapeDtypeStruct((_G, 128), jnp.float32),
        scratch_shapes=[
            pltpu.VMEM((_G, h3), jnp.float32),
            pltpu.VMEM((_G, 128), jnp.float32),
        ],
    )(agg3, b2, ids3d, wfc_p, bfc_p)
    return out[:, :1]


def kernel(x, edge_index, edge_attr, batch, W1, b1, W2, b2, W3, b3, Wfc, bfc):
    src = edge_index[0]
    dst = edge_index[1]
    loop = jnp.arange(_N, dtype=src.dtype)
    src_f = jnp.concatenate([src, loop])
    dst_f = jnp.concatenate([dst, loop])
    ew = jnp.concatenate([edge_attr, jnp.ones((_N,), edge_attr.dtype)])

    deg = jax.ops.segment_sum(ew, dst_f, num_segments=_N)
    dinv = jnp.where(deg > 0, 1.0 / jnp.sqrt(deg), 0.0)
    norm = dinv[src_f] * ew * dinv[dst_f]

    x_p = jnp.pad(x, ((0, _NPAD - _N), (0, 0)))

    def conv_agg(hw):
        msg = hw[src_f] * norm[:, None]
        return jax.ops.segment_sum(msg, dst_f, num_segments=_NPAD)

    hw1 = _mm(x_p, W1)
    agg1 = conv_agg(hw1)
    hw2 = _fused_mm(agg1, b1, W2)
    agg2 = conv_agg(hw2)
    hw3 = _fused_mm(agg2, b2, W3)
    agg3 = conv_agg(hw3)

    batch_p = jnp.pad(batch, (0, _NPAD - _N), constant_values=_G)
    return _pool_fc(agg3, b3, batch_p, Wfc, bfc)


# speedup vs baseline: 1.3532x; 1.3386x over previous
import functools

import jax
import jax.numpy as jnp
from jax.experimental import pallas as pl
from jax.experimental.pallas import tpu as pltpu

_N = 50000
_E = 800000
_G = 64
_BR = 512
_NPAD = 50176

_BN = 128
_NB = _NPAD // _BN
_C = 512
_EF = _E + _N
_NCH = -(-_EF // _C) + _NB + 1
_T = _NCH * _C


def _mm_kernel(x_ref, w_ref, o_ref):
    o_ref[...] = jnp.dot(x_ref[...], w_ref[...],
                         preferred_element_type=jnp.float32)


def _mm(x, w):
    m, k = x.shape
    n = w.shape[1]
    grid = (m // _BR,)
    return pl.pallas_call(
        _mm_kernel,
        grid=grid,
        in_specs=[
            pl.BlockSpec((_BR, k), lambda i: (i, 0)),
            pl.BlockSpec((k, n), lambda i: (0, 0)),
        ],
        out_specs=pl.BlockSpec((_BR, n), lambda i: (i, 0)),
        out_shape=jax.ShapeDtypeStruct((m, n), jnp.float32),
    )(x, w)


def _fused_kernel(a_ref, b_ref, w_ref, o_ref):
    h = jnp.maximum(a_ref[...] + b_ref[0:1, :], 0.0)
    o_ref[...] = jnp.dot(h, w_ref[...], preferred_element_type=jnp.float32)


def _fused_mm(agg, b, w):
    m, k = agg.shape
    n = w.shape[1]
    b2 = jnp.broadcast_to(b[None, :], (8, k))
    return pl.pallas_call(
        _fused_kernel,
        grid=(m // _BR,),
        in_specs=[
            pl.BlockSpec((_BR, k), lambda i: (i, 0)),
            pl.BlockSpec((8, k), lambda i: (0, 0)),
            pl.BlockSpec((k, n), lambda i: (0, 0)),
        ],
        out_specs=pl.BlockSpec((_BR, n), lambda i: (i, 0)),
        out_shape=jax.ShapeDtypeStruct((m, n), jnp.float32),
    )(agg, b2, w)


def _pool_kernel(a_ref, b_ref, ids_ref, wfc_ref, bfc_ref, o_ref,
                 sums_ref, cnt_ref):
    step = pl.program_id(0)
    nsteps = pl.num_programs(0)

    @pl.when(step == 0)
    def _init():
        sums_ref[...] = jnp.zeros_like(sums_ref)
        cnt_ref[...] = jnp.zeros_like(cnt_ref)

    h3 = jnp.maximum(a_ref[...] + b_ref[0:1, :], 0.0)
    ids = ids_ref[0]
    rows = jax.lax.broadcasted_iota(jnp.int32, (_G, _BR), 0)
    onehot = (ids == rows).astype(jnp.float32)
    sums_ref[...] += jnp.dot(onehot, h3, preferred_element_type=jnp.float32)
    cnt_ref[...] += jnp.broadcast_to(
        jnp.sum(onehot, axis=1, keepdims=True), (_G, 128))

    @pl.when(step == nsteps - 1)
    def _fin():
        h4 = sums_ref[...] / jnp.maximum(cnt_ref[:, 0:1], 1.0)
        logits = jnp.dot(h4, wfc_ref[...],
                         preferred_element_type=jnp.float32) + bfc_ref[0:1, :]
        o_ref[...] = jax.nn.sigmoid(logits)


def _pool_fc(agg3, b3, batch_p, wfc, bfc):
    m, h3 = agg3.shape
    ids3d = batch_p.reshape(m // _BR, 1, _BR)
    b2 = jnp.broadcast_to(b3[None, :], (8, h3))
    wfc_p = jnp.pad(wfc, ((0, 0), (0, 127)))
    bfc_p = jnp.broadcast_to(bfc[0], (8, 128))
    out = pl.pallas_call(
        _pool_kernel,
        grid=(m // _BR,),
        in_specs=[
            pl.BlockSpec((_BR, h3), lambda i: (i, 0)),
            pl.BlockSpec((8, h3), lambda i: (0, 0)),
            pl.BlockSpec((1, 1, _BR), lambda i: (i, 0, 0)),
            pl.BlockSpec((h3, 128), lambda i: (0, 0)),
            pl.BlockSpec((8, 128), lambda i: (0, 0)),
        ],
        out_specs=pl.BlockSpec((_G, 128), lambda i: (0, 0)),
        out_shape=jax.ShapeDtypeStruct((_G, 128), jnp.float32),
        scratch_shapes=[
            pltpu.VMEM((_G, h3), jnp.float32),
            pltpu.VMEM((_G, 128), jnp.float32),
        ],
    )(agg3, b2, ids3d, wfc_p, bfc_p)
    return out[:, :1]


def _seg_kernel(wj_ref, dst_ref, msg_ref, o_ref):
    j = pl.program_id(0)
    w = wj_ref[j]
    wprev = wj_ref[jnp.maximum(j - 1, 0)]
    first = jnp.logical_or(j == 0, w != wprev)
    ids = dst_ref[0]
    rows = jax.lax.broadcasted_iota(jnp.int32, (_BN, _C), 0)
    onehot = (ids == rows).astype(jnp.float32)
    contrib = jnp.dot(onehot, msg_ref[...],
                      preferred_element_type=jnp.float32)

    @pl.when(first)
    def _():
        o_ref[...] = contrib

    @pl.when(jnp.logical_not(first))
    def _():
        o_ref[...] += contrib


def _seg_reduce(wj, pdst3, msg):
    h = msg.shape[1]
    grid_spec = pltpu.PrefetchScalarGridSpec(
        num_scalar_prefetch=1,
        grid=(_NCH,),
        in_specs=[
            pl.BlockSpec((1, 1, _C), lambda j, wj_r: (j, 0, 0)),
            pl.BlockSpec((_C, h), lambda j, wj_r: (j, 0)),
        ],
        out_specs=pl.BlockSpec((_BN, h), lambda j, wj_r: (wj_r[j], 0)),
    )
    return pl.pallas_call(
        _seg_kernel,
        grid_spec=grid_spec,
        out_shape=jax.ShapeDtypeStruct((_NPAD, h), jnp.float32),
    )(wj, pdst3, msg)


def kernel(x, edge_index, edge_attr, batch, W1, b1, W2, b2, W3, b3, Wfc, bfc):
    src = edge_index[0]
    dst = edge_index[1]
    loop = jnp.arange(_N, dtype=src.dtype)
    src_f = jnp.concatenate([src, loop])
    dst_f = jnp.concatenate([dst, loop])
    ew = jnp.concatenate([edge_attr, jnp.ones((_N,), edge_attr.dtype)])

    deg = jax.ops.segment_sum(ew, dst_f, num_segments=_N)
    dinv = jnp.where(deg > 0, 1.0 / jnp.sqrt(deg), 0.0)
    norm = dinv[src_f] * ew * dinv[dst_f]

    x_p = jnp.pad(x, ((0, _NPAD - _N), (0, 0)))

    perm = jnp.argsort(dst_f)
    dst_s = dst_f[perm]
    src_s = src_f[perm]
    norm_s = norm[perm]
    bounds = (jnp.arange(_NB + 1, dtype=jnp.int32) * _BN)
    rp = jnp.searchsorted(dst_s, bounds).astype(jnp.int32)
    cnt = rp[1:] - rp[:-1]
    nch = jnp.maximum(1, -(-cnt // _C))
    firstc = jnp.concatenate([jnp.zeros((1,), jnp.int32),
                              jnp.cumsum(nch).astype(jnp.int32)])
    wj = jnp.repeat(jnp.arange(_NB, dtype=jnp.int32), nch,
                    total_repeat_length=_NCH)
    rj = jnp.arange(_NCH, dtype=jnp.int32) - firstc[wj]
    sj = rp[wj] + rj * _C
    endw = rp[wj + 1]
    eid = sj[:, None] + jnp.arange(_C, dtype=jnp.int32)[None, :]
    valid = eid < endw[:, None]
    eidc = jnp.clip(eid, 0, _EF - 1).reshape(_T)
    psrc = src_s[eidc]
    pnorm = jnp.where(valid.reshape(_T), norm_s[eidc], 0.0)
    pdst3 = jnp.where(valid, dst_s[eidc].reshape(_NCH, _C) - wj[:, None] * _BN,
                      -1).reshape(_NCH, 1, _C)

    def conv_agg(hw):
        msg = hw[psrc] * pnorm[:, None]
        return _seg_reduce(wj, pdst3, msg)

    hw1 = _mm(x_p, W1)
    agg1 = conv_agg(hw1)
    hw2 = _fused_mm(agg1, b1, W2)
    agg2 = conv_agg(hw2)
    hw3 = _fused_mm(agg2, b2, W3)
    agg3 = conv_agg(hw3)

    batch_p = jnp.pad(batch, (0, _NPAD - _N), constant_values=_G)
    return _pool_fc(agg3, b3, batch_p, Wfc, bfc)


# speedup vs baseline: 1.4217x; 1.0506x over previous
import functools

import jax
import jax.numpy as jnp
from jax.experimental import pallas as pl
from jax.experimental.pallas import tpu as pltpu

_N = 50000
_E = 800000
_G = 64
_BR = 512
_NPAD = 50176

_BN = 256
_NB = _NPAD // _BN
_C = 1024
_EF = _E + _N
_NCH = -(-_EF // _C) + _NB + 1
_T = _NCH * _C


def _mm_kernel(x_ref, w_ref, o_ref):
    o_ref[...] = jnp.dot(x_ref[...], w_ref[...],
                         preferred_element_type=jnp.float32)


def _mm(x, w):
    m, k = x.shape
    n = w.shape[1]
    grid = (m // _BR,)
    return pl.pallas_call(
        _mm_kernel,
        grid=grid,
        in_specs=[
            pl.BlockSpec((_BR, k), lambda i: (i, 0)),
            pl.BlockSpec((k, n), lambda i: (0, 0)),
        ],
        out_specs=pl.BlockSpec((_BR, n), lambda i: (i, 0)),
        out_shape=jax.ShapeDtypeStruct((m, n), jnp.float32),
    )(x, w)


def _fused_kernel(a_ref, b_ref, w_ref, o_ref):
    h = jnp.maximum(a_ref[...] + b_ref[0:1, :], 0.0)
    o_ref[...] = jnp.dot(h, w_ref[...], preferred_element_type=jnp.float32)


def _fused_mm(agg, b, w):
    m, k = agg.shape
    n = w.shape[1]
    b2 = jnp.broadcast_to(b[None, :], (8, k))
    return pl.pallas_call(
        _fused_kernel,
        grid=(m // _BR,),
        in_specs=[
            pl.BlockSpec((_BR, k), lambda i: (i, 0)),
            pl.BlockSpec((8, k), lambda i: (0, 0)),
            pl.BlockSpec((k, n), lambda i: (0, 0)),
        ],
        out_specs=pl.BlockSpec((_BR, n), lambda i: (i, 0)),
        out_shape=jax.ShapeDtypeStruct((m, n), jnp.float32),
    )(agg, b2, w)


def _pool_kernel(a_ref, b_ref, ids_ref, wfc_ref, bfc_ref, o_ref,
                 sums_ref, cnt_ref):
    step = pl.program_id(0)
    nsteps = pl.num_programs(0)

    @pl.when(step == 0)
    def _init():
        sums_ref[...] = jnp.zeros_like(sums_ref)
        cnt_ref[...] = jnp.zeros_like(cnt_ref)

    h3 = jnp.maximum(a_ref[...] + b_ref[0:1, :], 0.0)
    ids = ids_ref[0]
    rows = jax.lax.broadcasted_iota(jnp.int32, (_G, _BR), 0)
    onehot = (ids == rows).astype(jnp.float32)
    sums_ref[...] += jnp.dot(onehot, h3, preferred_element_type=jnp.float32)
    cnt_ref[...] += jnp.broadcast_to(
        jnp.sum(onehot, axis=1, keepdims=True), (_G, 128))

    @pl.when(step == nsteps - 1)
    def _fin():
        h4 = sums_ref[...] / jnp.maximum(cnt_ref[:, 0:1], 1.0)
        logits = jnp.dot(h4, wfc_ref[...],
                         preferred_element_type=jnp.float32) + bfc_ref[0:1, :]
        o_ref[...] = jax.nn.sigmoid(logits)


def _pool_fc(agg3, b3, batch_p, wfc, bfc):
    m, h3 = agg3.shape
    ids3d = batch_p.reshape(m // _BR, 1, _BR)
    b2 = jnp.broadcast_to(b3[None, :], (8, h3))
    wfc_p = jnp.pad(wfc, ((0, 0), (0, 127)))
    bfc_p = jnp.broadcast_to(bfc[0], (8, 128))
    out = pl.pallas_call(
        _pool_kernel,
        grid=(m // _BR,),
        in_specs=[
            pl.BlockSpec((_BR, h3), lambda i: (i, 0)),
            pl.BlockSpec((8, h3), lambda i: (0, 0)),
            pl.BlockSpec((1, 1, _BR), lambda i: (i, 0, 0)),
            pl.BlockSpec((h3, 128), lambda i: (0, 0)),
            pl.BlockSpec((8, 128), lambda i: (0, 0)),
        ],
        out_specs=pl.BlockSpec((_G, 128), lambda i: (0, 0)),
        out_shape=jax.ShapeDtypeStruct((_G, 128), jnp.float32),
        scratch_shapes=[
            pltpu.VMEM((_G, h3), jnp.float32),
            pltpu.VMEM((_G, 128), jnp.float32),
        ],
    )(agg3, b2, ids3d, wfc_p, bfc_p)
    return out[:, :1]


def _seg_kernel(wj_ref, dst_ref, msg_ref, o_ref):
    j = pl.program_id(0)
    w = wj_ref[j]
    wprev = wj_ref[jnp.maximum(j - 1, 0)]
    first = jnp.logical_or(j == 0, w != wprev)
    ids = dst_ref[0]
    rows = jax.lax.broadcasted_iota(jnp.int32, (_BN, _C), 0)
    onehot = (ids == rows).astype(jnp.float32)
    contrib = jnp.dot(onehot, msg_ref[...],
                      preferred_element_type=jnp.float32)

    @pl.when(first)
    def _():
        o_ref[...] = contrib

    @pl.when(jnp.logical_not(first))
    def _():
        o_ref[...] += contrib


def _seg_reduce(wj, pdst3, msg):
    h = msg.shape[1]
    grid_spec = pltpu.PrefetchScalarGridSpec(
        num_scalar_prefetch=1,
        grid=(_NCH,),
        in_specs=[
            pl.BlockSpec((1, 1, _C), lambda j, wj_r: (j, 0, 0)),
            pl.BlockSpec((_C, h), lambda j, wj_r: (j, 0)),
        ],
        out_specs=pl.BlockSpec((_BN, h), lambda j, wj_r: (wj_r[j], 0)),
    )
    return pl.pallas_call(
        _seg_kernel,
        grid_spec=grid_spec,
        out_shape=jax.ShapeDtypeStruct((_NPAD, h), jnp.float32),
    )(wj, pdst3, msg)


def kernel(x, edge_index, edge_attr, batch, W1, b1, W2, b2, W3, b3, Wfc, bfc):
    src = edge_index[0]
    dst = edge_index[1]
    loop = jnp.arange(_N, dtype=src.dtype)
    src_f = jnp.concatenate([src, loop])
    dst_f = jnp.concatenate([dst, loop])
    ew = jnp.concatenate([edge_attr, jnp.ones((_N,), edge_attr.dtype)])

    deg = jax.ops.segment_sum(ew, dst_f, num_segments=_N)
    dinv = jnp.where(deg > 0, 1.0 / jnp.sqrt(deg), 0.0)
    norm = dinv[src_f] * ew * dinv[dst_f]

    x_p = jnp.pad(x, ((0, _NPAD - _N), (0, 0)))

    perm = jnp.argsort(dst_f)
    dst_s = dst_f[perm]
    src_s = src_f[perm]
    norm_s = norm[perm]
    bounds = (jnp.arange(_NB + 1, dtype=jnp.int32) * _BN)
    rp = jnp.searchsorted(dst_s, bounds).astype(jnp.int32)
    cnt = rp[1:] - rp[:-1]
    nch = jnp.maximum(1, -(-cnt // _C))
    firstc = jnp.concatenate([jnp.zeros((1,), jnp.int32),
                              jnp.cumsum(nch).astype(jnp.int32)])
    wj = jnp.repeat(jnp.arange(_NB, dtype=jnp.int32), nch,
                    total_repeat_length=_NCH)
    rj = jnp.arange(_NCH, dtype=jnp.int32) - firstc[wj]
    sj = rp[wj] + rj * _C
    endw = rp[wj + 1]
    eid = sj[:, None] + jnp.arange(_C, dtype=jnp.int32)[None, :]
    valid = eid < endw[:, None]
    eidc = jnp.clip(eid, 0, _EF - 1).reshape(_T)
    psrc = src_s[eidc]
    pnorm = jnp.where(valid.reshape(_T), norm_s[eidc], 0.0)
    pdst3 = jnp.where(valid, dst_s[eidc].reshape(_NCH, _C) - wj[:, None] * _BN,
                      -1).reshape(_NCH, 1, _C)

    def conv_agg(hw):
        msg = hw[psrc] * pnorm[:, None]
        return _seg_reduce(wj, pdst3, msg)

    hw1 = _mm(x_p, W1)
    agg1 = conv_agg(hw1)
    hw2 = _fused_mm(agg1, b1, W2)
    agg2 = conv_agg(hw2)
    hw3 = _fused_mm(agg2, b2, W3)
    agg3 = conv_agg(hw3)

    batch_p = jnp.pad(batch, (0, _NPAD - _N), constant_values=_G)
    return _pool_fc(agg3, b3, batch_p, Wfc, bfc)
